# baseline (speedup 1.0000x reference)
import functools

import jax
import jax.numpy as jnp
from jax import lax
from jax.experimental import pallas as pl
from jax.experimental.pallas import tpu as pltpu
from jax.experimental.pallas import tpu_sc as plsc

N = 10000
E = 320000
H = 128
G = 8
C = 10

NC = 2
NS = 16
NW = NC * NS
BLK = 128
NBLK = 80
EPT = NBLK * BLK
E_PAD = EPT * NW
ROWS_PT = 640
N_PAD = NS * ROWS_PT
N_TAIL = N - (NS - 1) * ROWS_PT


def _edge_agg_sc(h, src_blocks, dst_blocks, zeros_blk):
    mesh = plsc.VectorSubcoreMesh(core_axis_name="c", subcore_axis_name="s")

    @functools.partial(
        pl.kernel,
        out_type=jax.ShapeDtypeStruct((NC, N, H), jnp.float32),
        mesh=mesh,
        scratch_types=[
            pltpu.VMEM((NBLK, BLK), jnp.int32),
            pltpu.VMEM((NBLK, BLK), jnp.int32),
            pltpu.VMEM((BLK, H), jnp.float32),
            pltpu.VMEM_SHARED((N_PAD, H), jnp.float32),
            pltpu.SemaphoreType.DMA,
        ],
    )
    def agg_kernel(h_hbm, src_hbm, dst_hbm, z_hbm, out_hbm,
                   src_v, dst_v, rows, acc_sh, sem_g):
        cid = lax.axis_index("c")
        sid = lax.axis_index("s")
        wid = cid * NS + sid

        pltpu.sync_copy(z_hbm, rows)
        for k in range(ROWS_PT // 128):
            pltpu.sync_copy(rows, acc_sh.at[pl.ds(sid * ROWS_PT + k * 128, 128)])
        plsc.subcore_barrier()

        pltpu.sync_copy(src_hbm.at[wid], src_v)
        pltpu.sync_copy(dst_hbm.at[wid], dst_v)

        def body(j, _):
            pltpu.async_copy(h_hbm.at[src_v.at[j]], rows, sem_g).wait()
            pltpu.sync_copy(rows, acc_sh.at[dst_v.at[j]], add=True)
            return ()

        lax.fori_loop(0, NBLK, body, ())
        plsc.subcore_barrier()

        @pl.when(sid < NS - 1)
        def _():
            pltpu.sync_copy(acc_sh.at[pl.ds(sid * ROWS_PT, ROWS_PT)],
                            out_hbm.at[cid, pl.ds(sid * ROWS_PT, ROWS_PT)])

        @pl.when(sid == NS - 1)
        def _():
            pltpu.sync_copy(acc_sh.at[pl.ds((NS - 1) * ROWS_PT, N_TAIL)],
                            out_hbm.at[cid, pl.ds((NS - 1) * ROWS_PT, N_TAIL)])

    return agg_kernel(h, src_blocks, dst_blocks, zeros_blk)


def _onehots(brow, bcol):
    oh = (lax.broadcasted_iota(jnp.int32, (G, N), 0) == brow).astype(jnp.float32)
    ohT = (lax.broadcasted_iota(jnp.int32, (N, G), 1) == bcol).astype(jnp.float32)
    return oh, ohT


def _graph_norm_body(x, oh, ohT, w, b, a):
    cnt = jnp.maximum(jnp.sum(oh, axis=1, keepdims=True), 1.0)
    mean = jnp.dot(oh, x, preferred_element_type=jnp.float32) / cnt
    sub = x - a * jnp.dot(ohT, mean, preferred_element_type=jnp.float32)
    var = jnp.dot(oh, sub * sub, preferred_element_type=jnp.float32) / cnt
    rstd = lax.rsqrt(var + 1e-5)
    return w * sub * jnp.dot(ohT, rstd, preferred_element_type=jnp.float32) + b


def _mm_t(x, w):
    return lax.dot_general(x, w, (((1,), (1,)), ((), ())),
                           preferred_element_type=jnp.float32)


def _norm_tc(x, brow, bcol, w, b, a):
    def body(x_ref, br_ref, bc_ref, w_ref, b_ref, a_ref, o_ref):
        oh, ohT = _onehots(br_ref[...], bc_ref[...])
        o_ref[...] = _graph_norm_body(x_ref[...], oh, ohT,
                                      w_ref[...], b_ref[...], a_ref[...])

    return pl.pallas_call(
        body, out_shape=jax.ShapeDtypeStruct((N, H), jnp.float32),
    )(x, brow, bcol, w, b, a)


def _conv_norm_tc(aggs, h, Wr, br, Wt, w, b, a, brow, bcol):
    def body(ag_ref, h_ref, Wr_ref, br_ref, Wt_ref, w_ref, b_ref, a_ref,
             brr_ref, bcc_ref, o_ref):
        agg = ag_ref[0] + ag_ref[1]
        conv = _mm_t(agg, Wr_ref[...]) + br_ref[...] + _mm_t(h_ref[...], Wt_ref[...])
        h1 = jnp.maximum(conv, 0.0)
        oh, ohT = _onehots(brr_ref[...], bcc_ref[...])
        o_ref[...] = _graph_norm_body(h1, oh, ohT,
                                      w_ref[...], b_ref[...], a_ref[...])

    return pl.pallas_call(
        body, out_shape=jax.ShapeDtypeStruct((N, H), jnp.float32),
    )(aggs, h, Wr, br, Wt, w, b, a, brow, bcol)


def _final_tc(aggs, h, Wr, br, Wt, brow, dW, db, oW, ob):
    def body(ag_ref, h_ref, Wr_ref, br_ref, Wt_ref, brr_ref,
             dW_ref, db_ref, oW_ref, ob_ref, o_ref):
        agg = ag_ref[0] + ag_ref[1]
        conv = _mm_t(agg, Wr_ref[...]) + br_ref[...] + _mm_t(h_ref[...], Wt_ref[...])
        h3 = jnp.maximum(conv, 0.0)
        oh = (lax.broadcasted_iota(jnp.int32, (G, N), 0) == brr_ref[...]
              ).astype(jnp.float32)
        cnt = jnp.maximum(jnp.sum(oh, axis=1, keepdims=True), 1.0)
        pooled = jnp.dot(oh, h3, preferred_element_type=jnp.float32) / cnt
        d = jnp.maximum(_mm_t(pooled, dW_ref[...]) + db_ref[...], 0.0)
        logits = _mm_t(d, oW_ref[...]) + ob_ref[...]
        m = jnp.max(logits, axis=1, keepdims=True)
        e = jnp.exp(logits - m)
        o_ref[...] = e / jnp.sum(e, axis=1, keepdims=True)

    return pl.pallas_call(
        body, out_shape=jax.ShapeDtypeStruct((G, C), jnp.float32),
    )(aggs, h, Wr, br, Wt, brow, dW, db, oW, ob)


def kernel(x, edge_index, batch, gn0_w, gn0_b, gn0_a, W1r, b1r, W1t,
           gn1_w, gn1_b, gn1_a, W2r, b2r, W2t, gn2_w, gn2_b, gn2_a,
           W3r, b3r, W3t, dW, db, oW, ob):
    pad = E_PAD - E
    src = jnp.concatenate([edge_index[0], jnp.zeros((pad,), jnp.int32)])
    dst = jnp.concatenate([edge_index[1], jnp.full((pad,), N, jnp.int32)])
    src_blocks = src.reshape(NW, NBLK, BLK)
    dst_blocks = dst.reshape(NW, NBLK, BLK)
    zeros_blk = jnp.zeros((128, H), jnp.float32)
    brow = batch.reshape(1, N)
    bcol = batch.reshape(N, 1)

    def row(v):
        return v.reshape(1, -1)

    h0 = _norm_tc(x, brow, bcol, row(gn0_w), row(gn0_b), row(gn0_a))
    a1 = _edge_agg_sc(h0, src_blocks, dst_blocks, zeros_blk)
    h1 = _conv_norm_tc(a1, h0, W1r, row(b1r), W1t,
                       row(gn1_w), row(gn1_b), row(gn1_a), brow, bcol)
    a2 = _edge_agg_sc(h1, src_blocks, dst_blocks, zeros_blk)
    h2 = _conv_norm_tc(a2, h1, W2r, row(b2r), W2t,
                       row(gn2_w), row(gn2_b), row(gn2_a), brow, bcol)
    a3 = _edge_agg_sc(h2, src_blocks, dst_blocks, zeros_blk)
    return _final_tc(a3, h2, W3r, row(b3r), W3t, brow, dW, row(db), oW, row(ob))


# speedup vs baseline: 1.6017x; 1.6017x over previous
import functools

import jax
import jax.numpy as jnp
from jax import lax
from jax.experimental import pallas as pl
from jax.experimental.pallas import tpu as pltpu
from jax.experimental.pallas import tpu_sc as plsc

N = 10000
E = 320000
H = 128
G = 8
C = 10

NC = 2
NS = 16
NW = NC * NS
BLK = 128
NBLK = -(-E // (NW * BLK))
EPT = NBLK * BLK
E_PAD = EPT * NW
ROWS_PT = 640
N_PAD = NS * ROWS_PT
N_TAIL = N - (NS - 1) * ROWS_PT


def _edge_agg_sc(h, src_blocks, dst_blocks, zeros_blk):
    mesh = plsc.VectorSubcoreMesh(core_axis_name="c", subcore_axis_name="s")

    @functools.partial(
        pl.kernel,
        out_type=jax.ShapeDtypeStruct((NC, N, H), jnp.float32),
        mesh=mesh,
        scratch_types=[
            pltpu.VMEM((NBLK, BLK), jnp.int32),
            pltpu.VMEM((NBLK, BLK), jnp.int32),
            pltpu.VMEM((BLK, H), jnp.float32),
            pltpu.VMEM_SHARED((N_PAD, H), jnp.float32),
            pltpu.SemaphoreType.DMA,
        ],
    )
    def agg_kernel(h_hbm, src_hbm, dst_hbm, z_hbm, out_hbm,
                   src_v, dst_v, rows, acc_sh, sem_g):
        cid = lax.axis_index("c")
        sid = lax.axis_index("s")
        wid = cid * NS + sid

        pltpu.sync_copy(z_hbm, rows)
        for k in range(ROWS_PT // 128):
            pltpu.sync_copy(rows, acc_sh.at[pl.ds(sid * ROWS_PT + k * 128, 128)])
        plsc.subcore_barrier()

        pltpu.sync_copy(src_hbm.at[wid], src_v)
        pltpu.sync_copy(dst_hbm.at[wid], dst_v)

        def body(j, _):
            pltpu.async_copy(h_hbm.at[src_v.at[j]], rows, sem_g).wait()
            pltpu.sync_copy(rows, acc_sh.at[dst_v.at[j]], add=True)
            return ()

        lax.fori_loop(0, NBLK, body, ())
        plsc.subcore_barrier()

        @pl.when(sid < NS - 1)
        def _():
            pltpu.sync_copy(acc_sh.at[pl.ds(sid * ROWS_PT, ROWS_PT)],
                            out_hbm.at[cid, pl.ds(sid * ROWS_PT, ROWS_PT)])

        @pl.when(sid == NS - 1)
        def _():
            pltpu.sync_copy(acc_sh.at[pl.ds((NS - 1) * ROWS_PT, N_TAIL)],
                            out_hbm.at[cid, pl.ds((NS - 1) * ROWS_PT, N_TAIL)])

    return agg_kernel(h, src_blocks, dst_blocks, zeros_blk)


def _onehots(brow, bcol):
    oh = (lax.broadcasted_iota(jnp.int32, (G, N), 0) == brow).astype(jnp.float32)
    ohT = (lax.broadcasted_iota(jnp.int32, (N, G), 1) == bcol).astype(jnp.float32)
    return oh, ohT


def _graph_norm_body(x, oh, ohT, w, b, a):
    cnt = jnp.maximum(jnp.sum(oh, axis=1, keepdims=True), 1.0)
    mean = jnp.dot(oh, x, preferred_element_type=jnp.float32) / cnt
    sub = x - a * jnp.dot(ohT, mean, preferred_element_type=jnp.float32)
    var = jnp.dot(oh, sub * sub, preferred_element_type=jnp.float32) / cnt
    rstd = lax.rsqrt(var + 1e-5)
    return w * sub * jnp.dot(ohT, rstd, preferred_element_type=jnp.float32) + b


def _mm_t(x, w):
    return lax.dot_general(x, w, (((1,), (1,)), ((), ())),
                           preferred_element_type=jnp.float32)


def _norm_tc(x, brow, bcol, w, b, a):
    def body(x_ref, br_ref, bc_ref, w_ref, b_ref, a_ref, o_ref):
        oh, ohT = _onehots(br_ref[...], bc_ref[...])
        o_ref[...] = _graph_norm_body(x_ref[...], oh, ohT,
                                      w_ref[...], b_ref[...], a_ref[...])

    return pl.pallas_call(
        body, out_shape=jax.ShapeDtypeStruct((N, H), jnp.float32),
    )(x, brow, bcol, w, b, a)


def _conv_norm_tc(aggs, h, Wr, br, Wt, w, b, a, brow, bcol):
    def body(ag_ref, h_ref, Wr_ref, br_ref, Wt_ref, w_ref, b_ref, a_ref,
             brr_ref, bcc_ref, o_ref):
        agg = ag_ref[0] + ag_ref[1]
        conv = _mm_t(agg, Wr_ref[...]) + br_ref[...] + _mm_t(h_ref[...], Wt_ref[...])
        h1 = jnp.maximum(conv, 0.0)
        oh, ohT = _onehots(brr_ref[...], bcc_ref[...])
        o_ref[...] = _graph_norm_body(h1, oh, ohT,
                                      w_ref[...], b_ref[...], a_ref[...])

    return pl.pallas_call(
        body, out_shape=jax.ShapeDtypeStruct((N, H), jnp.float32),
    )(aggs, h, Wr, br, Wt, w, b, a, brow, bcol)


def _final_tc(aggs, h, Wr, br, Wt, brow, dW, db, oW, ob):
    def body(ag_ref, h_ref, Wr_ref, br_ref, Wt_ref, brr_ref,
             dW_ref, db_ref, oW_ref, ob_ref, o_ref):
        agg = ag_ref[0] + ag_ref[1]
        conv = _mm_t(agg, Wr_ref[...]) + br_ref[...] + _mm_t(h_ref[...], Wt_ref[...])
        h3 = jnp.maximum(conv, 0.0)
        oh = (lax.broadcasted_iota(jnp.int32, (G, N), 0) == brr_ref[...]
              ).astype(jnp.float32)
        cnt = jnp.maximum(jnp.sum(oh, axis=1, keepdims=True), 1.0)
        pooled = jnp.dot(oh, h3, preferred_element_type=jnp.float32) / cnt
        d = jnp.maximum(_mm_t(pooled, dW_ref[...]) + db_ref[...], 0.0)
        logits = _mm_t(d, oW_ref[...]) + ob_ref[...]
        m = jnp.max(logits, axis=1, keepdims=True)
        e = jnp.exp(logits - m)
        o_ref[...] = e / jnp.sum(e, axis=1, keepdims=True)

    return pl.pallas_call(
        body, out_shape=jax.ShapeDtypeStruct((G, C), jnp.float32),
    )(aggs, h, Wr, br, Wt, brow, dW, db, oW, ob)


def kernel(x, edge_index, batch, gn0_w, gn0_b, gn0_a, W1r, b1r, W1t,
           gn1_w, gn1_b, gn1_a, W2r, b2r, W2t, gn2_w, gn2_b, gn2_a,
           W3r, b3r, W3t, dW, db, oW, ob):
    pad = E_PAD - E
    src = jnp.concatenate([edge_index[0], jnp.zeros((pad,), jnp.int32)])
    dst = jnp.concatenate([edge_index[1], jnp.full((pad,), N, jnp.int32)])
    src_blocks = src.reshape(NW, NBLK, BLK)
    dst_blocks = dst.reshape(NW, NBLK, BLK)
    zeros_blk = jnp.zeros((128, H), jnp.float32)
    brow = batch.reshape(1, N)
    bcol = batch.reshape(N, 1)

    def row(v):
        return v.reshape(1, -1)

    h0 = _norm_tc(x, brow, bcol, row(gn0_w), row(gn0_b), row(gn0_a))
    a1 = _edge_agg_sc(h0, src_blocks, dst_blocks, zeros_blk)
    h1 = _conv_norm_tc(a1, h0, W1r, row(b1r), W1t,
                       row(gn1_w), row(gn1_b), row(gn1_a), brow, bcol)
    a2 = _edge_agg_sc(h1, src_blocks, dst_blocks, zeros_blk)
    h2 = _conv_norm_tc(a2, h1, W2r, row(b2r), W2t,
                       row(gn2_w), row(gn2_b), row(gn2_a), brow, bcol)
    a3 = _edge_agg_sc(h2, src_blocks, dst_blocks, zeros_blk)
    return _final_tc(a3, h2, W3r, row(b3r), W3t, brow, dW, row(db), oW, row(ob))


# speedup vs baseline: 1.6027x; 1.0006x over previous
import functools

import jax
import jax.numpy as jnp
from jax import lax
from jax.experimental import pallas as pl
from jax.experimental.pallas import tpu as pltpu
from jax.experimental.pallas import tpu_sc as plsc

N = 10000
E = 320000
H = 128
G = 8
C = 10

NC = 2
NS = 16
NW = NC * NS
BLK = 128
NBLK = -(-E // (NW * BLK))
EPT = NBLK * BLK
E_PAD = EPT * NW
ROWS_PT = 640
N_PAD = NS * ROWS_PT
N_TAIL = N - (NS - 1) * ROWS_PT


def _edge_agg_sc(h, src_blocks, dst_blocks, zeros_blk):
    mesh = plsc.VectorSubcoreMesh(core_axis_name="c", subcore_axis_name="s")

    @functools.partial(
        pl.kernel,
        out_type=jax.ShapeDtypeStruct((NC, N, H), jnp.float32),
        mesh=mesh,
        scratch_types=[
            pltpu.VMEM((NBLK, BLK), jnp.int32),
            pltpu.VMEM((NBLK, BLK), jnp.int32),
            pltpu.VMEM((BLK, H), jnp.float32),
            pltpu.VMEM_SHARED((N_PAD, H), jnp.float32),
            pltpu.SemaphoreType.DMA,
        ],
    )
    def agg_kernel(h_hbm, src_hbm, dst_hbm, z_hbm, out_hbm,
                   src_v, dst_v, rows, acc_sh, sem_g):
        cid = lax.axis_index("c")
        sid = lax.axis_index("s")
        wid = cid * NS + sid

        pltpu.sync_copy(z_hbm, rows)
        for k in range(ROWS_PT // 128):
            pltpu.sync_copy(rows, acc_sh.at[pl.ds(sid * ROWS_PT + k * 128, 128)])
        plsc.subcore_barrier()

        pltpu.sync_copy(src_hbm.at[wid], src_v)
        pltpu.sync_copy(dst_hbm.at[wid], dst_v)

        def body(j, _):
            pltpu.async_copy(h_hbm.at[src_v.at[j]], rows, sem_g).wait()
            pltpu.sync_copy(rows, acc_sh.at[dst_v.at[j]], add=True)
            return ()

        lax.fori_loop(0, NBLK, body, ())
        plsc.subcore_barrier()

        @pl.when(sid < NS - 1)
        def _():
            pltpu.sync_copy(acc_sh.at[pl.ds(sid * ROWS_PT, ROWS_PT)],
                            out_hbm.at[cid, pl.ds(sid * ROWS_PT, ROWS_PT)])

        @pl.when(sid == NS - 1)
        def _():
            pltpu.sync_copy(acc_sh.at[pl.ds((NS - 1) * ROWS_PT, N_TAIL)],
                            out_hbm.at[cid, pl.ds((NS - 1) * ROWS_PT, N_TAIL)])

    return agg_kernel(h, src_blocks, dst_blocks, zeros_blk)


def _onehots(brow, bcol):
    oh = (lax.broadcasted_iota(jnp.int32, (G, N), 0) == brow).astype(jnp.float32)
    ohT = (lax.broadcasted_iota(jnp.int32, (N, G), 1) == bcol).astype(jnp.float32)
    return oh, ohT


def _graph_norm_body(x, oh, ohT, w, b, a):
    cnt = jnp.maximum(jnp.sum(oh, axis=1, keepdims=True), 1.0)
    mean = jnp.dot(oh, x, preferred_element_type=jnp.float32) / cnt
    sub = x - a * jnp.dot(ohT, mean, preferred_element_type=jnp.float32)
    var = jnp.dot(oh, sub * sub, preferred_element_type=jnp.float32) / cnt
    rstd = lax.rsqrt(var + 1e-5)
    return w * sub * jnp.dot(ohT, rstd, preferred_element_type=jnp.float32) + b


def _mm_t(x, w):
    return lax.dot_general(x, w, (((1,), (1,)), ((), ())),
                           preferred_element_type=jnp.float32)


def _norm_tc(x, brow, bcol, w, b, a):
    def body(x_ref, br_ref, bc_ref, w_ref, b_ref, a_ref, o_ref):
        oh, ohT = _onehots(br_ref[...], bc_ref[...])
        o_ref[...] = _graph_norm_body(x_ref[...], oh, ohT,
                                      w_ref[...], b_ref[...], a_ref[...])

    return pl.pallas_call(
        body, out_shape=jax.ShapeDtypeStruct((N, H), jnp.float32),
    )(x, brow, bcol, w, b, a)


def _conv_norm_tc(aggs, h, Wr, br, Wt, w, b, a, brow, bcol):
    def body(ag_ref, h_ref, Wr_ref, br_ref, Wt_ref, w_ref, b_ref, a_ref,
             brr_ref, bcc_ref, o_ref):
        agg = ag_ref[0] + ag_ref[1]
        conv = _mm_t(agg, Wr_ref[...]) + br_ref[...] + _mm_t(h_ref[...], Wt_ref[...])
        h1 = jnp.maximum(conv, 0.0)
        oh, ohT = _onehots(brr_ref[...], bcc_ref[...])
        o_ref[...] = _graph_norm_body(h1, oh, ohT,
                                      w_ref[...], b_ref[...], a_ref[...])

    return pl.pallas_call(
        body, out_shape=jax.ShapeDtypeStruct((N, H), jnp.float32),
    )(aggs, h, Wr, br, Wt, w, b, a, brow, bcol)


def _final_tc(aggs, h, Wr, br, Wt, brow, dW, db, oW, ob):
    def body(ag_ref, h_ref, Wr_ref, br_ref, Wt_ref, brr_ref,
             dW_ref, db_ref, oW_ref, ob_ref, o_ref):
        agg = ag_ref[0] + ag_ref[1]
        conv = _mm_t(agg, Wr_ref[...]) + br_ref[...] + _mm_t(h_ref[...], Wt_ref[...])
        h3 = jnp.maximum(conv, 0.0)
        oh = (lax.broadcasted_iota(jnp.int32, (G, N), 0) == brr_ref[...]
              ).astype(jnp.float32)
        cnt = jnp.maximum(jnp.sum(oh, axis=1, keepdims=True), 1.0)
        pooled = jnp.dot(oh, h3, preferred_element_type=jnp.float32) / cnt
        d = jnp.maximum(_mm_t(pooled, dW_ref[...]) + db_ref[...], 0.0)
        logits = _mm_t(d, oW_ref[...]) + ob_ref[...]
        m = jnp.max(logits, axis=1, keepdims=True)
        e = jnp.exp(logits - m)
        o_ref[...] = e / jnp.sum(e, axis=1, keepdims=True)

    return pl.pallas_call(
        body, out_shape=jax.ShapeDtypeStruct((G, C), jnp.float32),
    )(aggs, h, Wr, br, Wt, brow, dW, db, oW, ob)


def kernel(x, edge_index, batch, gn0_w, gn0_b, gn0_a, W1r, b1r, W1t,
           gn1_w, gn1_b, gn1_a, W2r, b2r, W2t, gn2_w, gn2_b, gn2_a,
           W3r, b3r, W3t, dW, db, oW, ob):
    pad = E_PAD - E
    src = jnp.concatenate([edge_index[0], jnp.zeros((pad,), jnp.int32)])
    pad_dst = N + (jnp.arange(pad, dtype=jnp.int32) % (N_PAD - N))
    dst = jnp.concatenate([edge_index[1], pad_dst])
    src_blocks = src.reshape(NW, NBLK, BLK)
    dst_blocks = dst.reshape(NW, NBLK, BLK)
    zeros_blk = jnp.zeros((128, H), jnp.float32)
    brow = batch.reshape(1, N)
    bcol = batch.reshape(N, 1)

    def row(v):
        return v.reshape(1, -1)

    h0 = _norm_tc(x, brow, bcol, row(gn0_w), row(gn0_b), row(gn0_a))
    a1 = _edge_agg_sc(h0, src_blocks, dst_blocks, zeros_blk)
    h1 = _conv_norm_tc(a1, h0, W1r, row(b1r), W1t,
                       row(gn1_w), row(gn1_b), row(gn1_a), brow, bcol)
    a2 = _edge_agg_sc(h1, src_blocks, dst_blocks, zeros_blk)
    h2 = _conv_norm_tc(a2, h1, W2r, row(b2r), W2t,
                       row(gn2_w), row(gn2_b), row(gn2_a), brow, bcol)
    a3 = _edge_agg_sc(h2, src_blocks, dst_blocks, zeros_blk)
    return _final_tc(a3, h2, W3r, row(b3r), W3t, brow, dW, row(db), oW, row(ob))


# speedup vs baseline: 2.1583x; 1.3467x over previous
import functools

import jax
import jax.numpy as jnp
from jax import lax
from jax.experimental import pallas as pl
from jax.experimental.pallas import tpu as pltpu
from jax.experimental.pallas import tpu_sc as plsc

N = 10000
E = 320000
H = 128
G = 8
C = 10

NC = 2
NS = 16
NW = NC * NS
BLK = 128
NBLK0 = 50
NBLK1 = 107
NBLKMX = NBLK1
E0 = NS * NBLK0 * BLK
E1_PAD = NS * NBLK1 * BLK
ROWS_PT = 640
N_PAD = NS * ROWS_PT
N_TAIL = N - (NS - 1) * ROWS_PT


def _edge_agg_sc(h, src_blocks, dst_blocks, zeros_blk):
    mesh = plsc.VectorSubcoreMesh(core_axis_name="c", subcore_axis_name="s")

    @functools.partial(
        pl.kernel,
        out_type=jax.ShapeDtypeStruct((NC, N, H), jnp.float32),
        mesh=mesh,
        scratch_types=[
            pltpu.VMEM((NBLKMX, BLK), jnp.int32),
            pltpu.VMEM((NBLKMX, BLK), jnp.int32),
            pltpu.VMEM((BLK, H), jnp.float32),
            pltpu.VMEM_SHARED((N_PAD, H), jnp.float32),
            pltpu.SemaphoreType.DMA,
        ],
    )
    def agg_kernel(h_hbm, src_hbm, dst_hbm, z_hbm, out_hbm,
                   src_v, dst_v, rows, acc_sh, sem_g):
        cid = lax.axis_index("c")
        sid = lax.axis_index("s")
        wid = cid * NS + sid

        pltpu.sync_copy(z_hbm, rows)
        for k in range(ROWS_PT // 128):
            pltpu.sync_copy(rows, acc_sh.at[pl.ds(sid * ROWS_PT + k * 128, 128)])
        plsc.subcore_barrier()

        pltpu.sync_copy(src_hbm.at[wid], src_v)
        pltpu.sync_copy(dst_hbm.at[wid], dst_v)

        def body(j, _):
            pltpu.async_copy(h_hbm.at[src_v.at[j]], rows, sem_g).wait()
            pltpu.sync_copy(rows, acc_sh.at[dst_v.at[j]], add=True)
            return ()

        nblk = jnp.where(cid == 0, NBLK0, NBLK1)
        lax.fori_loop(0, nblk, body, ())
        plsc.subcore_barrier()

        @pl.when(sid < NS - 1)
        def _():
            pltpu.sync_copy(acc_sh.at[pl.ds(sid * ROWS_PT, ROWS_PT)],
                            out_hbm.at[cid, pl.ds(sid * ROWS_PT, ROWS_PT)])

        @pl.when(sid == NS - 1)
        def _():
            pltpu.sync_copy(acc_sh.at[pl.ds((NS - 1) * ROWS_PT, N_TAIL)],
                            out_hbm.at[cid, pl.ds((NS - 1) * ROWS_PT, N_TAIL)])

    return agg_kernel(h, src_blocks, dst_blocks, zeros_blk)


def _onehots(brow, bcol):
    oh = (lax.broadcasted_iota(jnp.int32, (G, N), 0) == brow).astype(jnp.float32)
    ohT = (lax.broadcasted_iota(jnp.int32, (N, G), 1) == bcol).astype(jnp.float32)
    return oh, ohT


def _graph_norm_body(x, oh, ohT, w, b, a):
    cnt = jnp.maximum(jnp.sum(oh, axis=1, keepdims=True), 1.0)
    mean = jnp.dot(oh, x, preferred_element_type=jnp.float32) / cnt
    sub = x - a * jnp.dot(ohT, mean, preferred_element_type=jnp.float32)
    var = jnp.dot(oh, sub * sub, preferred_element_type=jnp.float32) / cnt
    rstd = lax.rsqrt(var + 1e-5)
    return w * sub * jnp.dot(ohT, rstd, preferred_element_type=jnp.float32) + b


def _mm_t(x, w):
    return lax.dot_general(x, w, (((1,), (1,)), ((), ())),
                           preferred_element_type=jnp.float32)


def _norm_tc(x, brow, bcol, w, b, a):
    def body(x_ref, br_ref, bc_ref, w_ref, b_ref, a_ref, o_ref):
        oh, ohT = _onehots(br_ref[...], bc_ref[...])
        o_ref[...] = _graph_norm_body(x_ref[...], oh, ohT,
                                      w_ref[...], b_ref[...], a_ref[...])

    return pl.pallas_call(
        body, out_shape=jax.ShapeDtypeStruct((N, H), jnp.float32),
    )(x, brow, bcol, w, b, a)


def _conv_norm_tc(aggs, h, Wr, br, Wt, w, b, a, brow, bcol):
    def body(ag_ref, h_ref, Wr_ref, br_ref, Wt_ref, w_ref, b_ref, a_ref,
             brr_ref, bcc_ref, o_ref):
        agg = ag_ref[0] + ag_ref[1]
        conv = _mm_t(agg, Wr_ref[...]) + br_ref[...] + _mm_t(h_ref[...], Wt_ref[...])
        h1 = jnp.maximum(conv, 0.0)
        oh, ohT = _onehots(brr_ref[...], bcc_ref[...])
        o_ref[...] = _graph_norm_body(h1, oh, ohT,
                                      w_ref[...], b_ref[...], a_ref[...])

    return pl.pallas_call(
        body, out_shape=jax.ShapeDtypeStruct((N, H), jnp.float32),
    )(aggs, h, Wr, br, Wt, w, b, a, brow, bcol)


def _final_tc(aggs, h, Wr, br, Wt, brow, dW, db, oW, ob):
    def body(ag_ref, h_ref, Wr_ref, br_ref, Wt_ref, brr_ref,
             dW_ref, db_ref, oW_ref, ob_ref, o_ref):
        agg = ag_ref[0] + ag_ref[1]
        conv = _mm_t(agg, Wr_ref[...]) + br_ref[...] + _mm_t(h_ref[...], Wt_ref[...])
        h3 = jnp.maximum(conv, 0.0)
        oh = (lax.broadcasted_iota(jnp.int32, (G, N), 0) == brr_ref[...]
              ).astype(jnp.float32)
        cnt = jnp.maximum(jnp.sum(oh, axis=1, keepdims=True), 1.0)
        pooled = jnp.dot(oh, h3, preferred_element_type=jnp.float32) / cnt
        d = jnp.maximum(_mm_t(pooled, dW_ref[...]) + db_ref[...], 0.0)
        logits = _mm_t(d, oW_ref[...]) + ob_ref[...]
        m = jnp.max(logits, axis=1, keepdims=True)
        e = jnp.exp(logits - m)
        o_ref[...] = e / jnp.sum(e, axis=1, keepdims=True)

    return pl.pallas_call(
        body, out_shape=jax.ShapeDtypeStruct((G, C), jnp.float32),
    )(aggs, h, Wr, br, Wt, brow, dW, db, oW, ob)


def kernel(x, edge_index, batch, gn0_w, gn0_b, gn0_a, W1r, b1r, W1t,
           gn1_w, gn1_b, gn1_a, W2r, b2r, W2t, gn2_w, gn2_b, gn2_a,
           W3r, b3r, W3t, dW, db, oW, ob):
    pad1 = E1_PAD - (E - E0)
    tail0 = NBLKMX - NBLK0
    pad_dst = N + (jnp.arange(pad1, dtype=jnp.int32) % (N_PAD - N))
    src0 = jnp.concatenate(
        [edge_index[0, :E0].reshape(NS, NBLK0, BLK),
         jnp.zeros((NS, tail0, BLK), jnp.int32)], axis=1)
    src1 = jnp.concatenate(
        [edge_index[0, E0:], jnp.zeros((pad1,), jnp.int32)]).reshape(NS, NBLK1, BLK)
    dst0 = jnp.concatenate(
        [edge_index[1, :E0].reshape(NS, NBLK0, BLK),
         jnp.full((NS, tail0, BLK), N, jnp.int32)], axis=1)
    dst1 = jnp.concatenate(
        [edge_index[1, E0:], pad_dst]).reshape(NS, NBLK1, BLK)
    src_blocks = jnp.concatenate([src0, src1], axis=0)
    dst_blocks = jnp.concatenate([dst0, dst1], axis=0)
    zeros_blk = jnp.zeros((128, H), jnp.float32)
    brow = batch.reshape(1, N)
    bcol = batch.reshape(N, 1)

    def row(v):
        return v.reshape(1, -1)

    h0 = _norm_tc(x, brow, bcol, row(gn0_w), row(gn0_b), row(gn0_a))
    a1 = _edge_agg_sc(h0, src_blocks, dst_blocks, zeros_blk)
    h1 = _conv_norm_tc(a1, h0, W1r, row(b1r), W1t,
                       row(gn1_w), row(gn1_b), row(gn1_a), brow, bcol)
    a2 = _edge_agg_sc(h1, src_blocks, dst_blocks, zeros_blk)
    h2 = _conv_norm_tc(a2, h1, W2r, row(b2r), W2t,
                       row(gn2_w), row(gn2_b), row(gn2_a), brow, bcol)
    a3 = _edge_agg_sc(h2, src_blocks, dst_blocks, zeros_blk)
    return _final_tc(a3, h2, W3r, row(b3r), W3t, brow, dW, row(db), oW, row(ob))
